# baseline (speedup 1.0000x reference)
import jax
import jax.numpy as jnp
from jax.experimental import pallas as pl
from jax.experimental.pallas import tpu as pltpu

_N = 50000
_D = 256
_H = 128
_BLOCK = 2000


def _fused_body(inp_ref, xr_ref, w1_ref, b1_ref, w2c_ref, b2_ref, pe_ref,
                out_ref):
    x = inp_ref[...] + pe_ref[...]
    h = jnp.dot(x, w1_ref[...], preferred_element_type=jnp.float32,
                precision=jax.lax.Precision.DEFAULT)
    h = jnp.maximum(h + b1_ref[...], 0.0)
    c = jnp.dot(h, w2c_ref[...], preferred_element_type=jnp.float32,
                precision=jax.lax.Precision.DEFAULT)
    std = jnp.maximum(c[:, :_H] + b2_ref[0], 0.0)
    mean = jnp.maximum(c[:, _H:] + b2_ref[1], 0.0)
    xr = xr_ref[...].reshape(_BLOCK, 1)
    gate = jax.nn.sigmoid(xr * std + mean)
    out_ref[:, :_H] = x[:, :_H] * gate
    out_ref[:, _H:] = x[:, _H:] * gate


@jax.jit
def _run(input, t, W1, b1, W2, b2, pe, pe_coff, x_rand):
    pe_row = pe_coff * jax.lax.dynamic_slice_in_dim(pe, t + 1, 1, axis=0)
    b1r = b1.reshape(1, _D)
    w2c = jnp.concatenate(
        [jnp.broadcast_to(W2[:, 0:1], (_D, _H)),
         jnp.broadcast_to(W2[:, 1:2], (_D, _H))], axis=1)
    nblk = _N // _BLOCK
    xr3 = x_rand.reshape(nblk, 1, _BLOCK)
    return pl.pallas_call(
        _fused_body,
        grid=(nblk,),
        in_specs=[
            pl.BlockSpec((_BLOCK, _D), lambda i: (i, 0)),
            pl.BlockSpec((1, 1, _BLOCK), lambda i: (i, 0, 0)),
            pl.BlockSpec((_D, _D), lambda i: (0, 0)),
            pl.BlockSpec((1, _D), lambda i: (0, 0)),
            pl.BlockSpec((_D, _D), lambda i: (0, 0)),
            pl.BlockSpec(memory_space=pltpu.SMEM),
            pl.BlockSpec((1, _D), lambda i: (0, 0)),
        ],
        out_specs=pl.BlockSpec((_BLOCK, _D), lambda i: (i, 0)),
        out_shape=jax.ShapeDtypeStruct((_N, _D), jnp.float32),
        compiler_params=pltpu.CompilerParams(
            dimension_semantics=("parallel",)),
    )(input, xr3, W1, b1r, w2c, b2, pe_row)


def kernel(graph, input, t, W1, b1, W2, b2, pe, pe_coff, x_rand):
    return _run(input, t, W1, b1, W2, b2, pe, pe_coff, x_rand)


# speedup vs baseline: 1.2033x; 1.2033x over previous
import jax
import jax.numpy as jnp
from jax.experimental import pallas as pl
from jax.experimental.pallas import tpu as pltpu

_N = 50000
_D = 256
_H = 128
_BLOCK = 10000


def _fused_body(inp_ref, xr_ref, w1_ref, b1_ref, w2c_ref, b2_ref, pe_ref,
                out_ref):
    x = inp_ref[...] + pe_ref[...]
    h = jnp.dot(x, w1_ref[...], preferred_element_type=jnp.float32,
                precision=jax.lax.Precision.DEFAULT)
    h = jnp.maximum(h + b1_ref[...], 0.0)
    c = jnp.dot(h, w2c_ref[...], preferred_element_type=jnp.float32,
                precision=jax.lax.Precision.DEFAULT)
    std = jnp.maximum(c[:, :_H] + b2_ref[0], 0.0)
    mean = jnp.maximum(c[:, _H:] + b2_ref[1], 0.0)
    xr = xr_ref[...].reshape(_BLOCK, 1)
    gate = jax.nn.sigmoid(xr * std + mean)
    out_ref[:, :_H] = x[:, :_H] * gate
    out_ref[:, _H:] = x[:, _H:] * gate


@jax.jit
def _run(input, t, W1, b1, W2, b2, pe, pe_coff, x_rand):
    pe_row = pe_coff * jax.lax.dynamic_slice_in_dim(pe, t + 1, 1, axis=0)
    b1r = b1.reshape(1, _D)
    w2c = jnp.concatenate(
        [jnp.broadcast_to(W2[:, 0:1], (_D, _H)),
         jnp.broadcast_to(W2[:, 1:2], (_D, _H))], axis=1)
    nblk = _N // _BLOCK
    xr3 = x_rand.reshape(nblk, 1, _BLOCK)
    return pl.pallas_call(
        _fused_body,
        grid=(nblk,),
        in_specs=[
            pl.BlockSpec((_BLOCK, _D), lambda i: (i, 0)),
            pl.BlockSpec((1, 1, _BLOCK), lambda i: (i, 0, 0)),
            pl.BlockSpec((_D, _D), lambda i: (0, 0)),
            pl.BlockSpec((1, _D), lambda i: (0, 0)),
            pl.BlockSpec((_D, _D), lambda i: (0, 0)),
            pl.BlockSpec(memory_space=pltpu.SMEM),
            pl.BlockSpec((1, _D), lambda i: (0, 0)),
        ],
        out_specs=pl.BlockSpec((_BLOCK, _D), lambda i: (i, 0)),
        out_shape=jax.ShapeDtypeStruct((_N, _D), jnp.float32),
        compiler_params=pltpu.CompilerParams(
            dimension_semantics=("parallel",)),
    )(input, xr3, W1, b1r, w2c, b2, pe_row)


def kernel(graph, input, t, W1, b1, W2, b2, pe, pe_coff, x_rand):
    return _run(input, t, W1, b1, W2, b2, pe, pe_coff, x_rand)
